# baseline (speedup 1.0000x reference)
import functools

import jax
import jax.numpy as jnp
from jax import lax
from jax.experimental import pallas as pl
from jax.experimental.pallas import tpu as pltpu
from jax.experimental.pallas import tpu_sc as plsc

BATCH = 16384
EMBED_DIM = 64
PACKED = 2 * EMBED_DIM

NC = 2
NS = 16
NW = NC * NS
BPW = BATCH // NW
CHUNK = 128
CPW = BPW // CHUNK

_sc_mesh = plsc.VectorSubcoreMesh(core_axis_name="c", subcore_axis_name="s")


@jax.jit
def _sc_gather(user_packed, product_packed, uidx, pidx):
    @functools.partial(
        pl.kernel,
        mesh=_sc_mesh,
        out_type=(
            jax.ShapeDtypeStruct((BATCH, PACKED), jnp.float32),
            jax.ShapeDtypeStruct((BATCH, PACKED), jnp.float32),
        ),
        scratch_types=[
            pltpu.VMEM((CPW, CHUNK), jnp.int32),
            pltpu.VMEM((CPW, CHUNK), jnp.int32),
            pltpu.VMEM((CHUNK, PACKED), jnp.float32),
            pltpu.VMEM((CHUNK, PACKED), jnp.float32),
            pltpu.SemaphoreType.DMA,
        ],
    )
    def k(ut_hbm, pt_hbm, ui_hbm, pi_hbm, ue_hbm, pe_hbm,
          ui_v, pi_v, ru_v, rp_v, sem):
        wid = lax.axis_index("s") * NC + lax.axis_index("c")
        row0 = wid * CPW
        base = wid * BPW
        pltpu.sync_copy(ui_hbm.at[pl.ds(row0, CPW)], ui_v)
        pltpu.sync_copy(pi_hbm.at[pl.ds(row0, CPW)], pi_v)
        for c in range(CPW):
            cu = pltpu.async_copy(ut_hbm.at[ui_v.at[c]], ru_v, sem)
            cp = pltpu.async_copy(pt_hbm.at[pi_v.at[c]], rp_v, sem)
            cu.wait()
            cp.wait()
            pltpu.sync_copy(ru_v, ue_hbm.at[pl.ds(base + c * CHUNK, CHUNK)])
            pltpu.sync_copy(rp_v, pe_hbm.at[pl.ds(base + c * CHUNK, CHUNK)])

    return k(user_packed, product_packed, uidx, pidx)


def _mlp_body(bu, bp, su, sp, uf, bd, w1, b1, w2, b2, w3a, w3b, w3c, b3,
              w4, b4, out):
    bu_ = bu[...]
    bp_ = bp[...]
    ue = jnp.where(su[...] > 0, bu_[:, EMBED_DIM:], bu_[:, :EMBED_DIM])
    pe = jnp.where(sp[...] > 0, bp_[:, EMBED_DIM:], bp_[:, :EMBED_DIM])
    m = ue * pe
    ufeat = jnp.maximum(
        jnp.dot(uf[...], w1[...], preferred_element_type=jnp.float32) + b1[...], 0.0)
    bfeat = jnp.maximum(
        jnp.dot(bd[...], w2[...], preferred_element_type=jnp.float32) + b2[...], 0.0)
    h = (jnp.dot(m, w3a[...], preferred_element_type=jnp.float32)
         + jnp.dot(ufeat, w3b[...], preferred_element_type=jnp.float32)
         + jnp.dot(bfeat, w3c[...], preferred_element_type=jnp.float32)
         + b3[...])
    h = jnp.maximum(h, 0.0)
    logit = jnp.dot(h, w4[...], preferred_element_type=jnp.float32) + b4[...]
    out[...] = jax.nn.sigmoid(logit)


_TC_BLOCK = 2048


@jax.jit
def _tc_mlp(bu, bp, su, sp, uf, bd, w1, b1, w2, b2, w3a, w3b, w3c, b3, w4, b4):
    def row_block(width):
        return pl.BlockSpec((_TC_BLOCK, width), lambda i: (i, 0))

    def whole(a):
        return pl.BlockSpec(a.shape, lambda i: (0, 0))

    return pl.pallas_call(
        _mlp_body,
        grid=(BATCH // _TC_BLOCK,),
        in_specs=[row_block(PACKED), row_block(PACKED), row_block(1),
                  row_block(1), row_block(11), row_block(3),
                  whole(w1), whole(b1), whole(w2), whole(b2),
                  whole(w3a), whole(w3b), whole(w3c), whole(b3),
                  whole(w4), whole(b4)],
        out_specs=row_block(1),
        out_shape=jax.ShapeDtypeStruct((BATCH, 1), jnp.float32),
    )(bu, bp, su, sp, uf, bd, w1, b1, w2, b2, w3a, w3b, w3c, b3, w4, b4)


def kernel(user_ids, product_ids, user_features, behavior_data,
           user_table, product_table, W1, b1, W2, b2, W3, b3, W4, b4):
    uq = (user_ids >> 1).reshape(BATCH // CHUNK, CHUNK)
    pq = (product_ids >> 1).reshape(BATCH // CHUNK, CHUNK)
    su = (user_ids & 1).astype(jnp.float32).reshape(BATCH, 1)
    sp = (product_ids & 1).astype(jnp.float32).reshape(BATCH, 1)
    up = user_table.reshape(user_table.shape[0] // 2, PACKED)
    pp = product_table.reshape(product_table.shape[0] // 2, PACKED)
    bu, bp = _sc_gather(up, pp, uq, pq)
    return _tc_mlp(
        bu, bp, su, sp, user_features, behavior_data,
        W1.T, b1.reshape(1, 32), W2.T, b2.reshape(1, 32),
        W3[:, :EMBED_DIM].T, W3[:, EMBED_DIM:EMBED_DIM + 32].T,
        W3[:, EMBED_DIM + 32:].T, b3.reshape(1, 32),
        W4.T, b4.reshape(1, 1))
